# baseline (speedup 1.0000x reference)
import functools

import jax
import jax.numpy as jnp
from jax import lax
from jax.experimental import pallas as pl
from jax.experimental.pallas import tpu as pltpu
from jax.experimental.pallas import tpu_sc as plsc

VOCAB = 1000
EMBED_DIM = 128
BATCH = 1024
SEQ = 50

NC, NS = 2, 16
NW = NC * NS
RPW = BATCH // NW

CHUNKS = ((0, 12), (12, 18), (30, 20))

_SC_MESH = plsc.VectorSubcoreMesh(
    core_axis_name="c", subcore_axis_name="s", num_cores=NC, num_subcores=NS)


def _make_sc_embed(s0, sch):
    @functools.partial(
        pl.kernel,
        out_type=jax.ShapeDtypeStruct((sch, BATCH, EMBED_DIM), jnp.float32),
        mesh=_SC_MESH,
        scratch_types=[
            pltpu.VMEM((RPW, SEQ), jnp.int32),
            pltpu.VMEM((sch, RPW), jnp.int32),
            pltpu.VMEM((2, RPW, EMBED_DIM), jnp.float32),
            pltpu.SemaphoreType.DMA,
            pltpu.SemaphoreType.DMA,
            pltpu.SemaphoreType.DMA,
            pltpu.SemaphoreType.DMA,
        ],
        compiler_params=pltpu.CompilerParams(
            use_tc_tiling_on_sc=True, needs_layout_passes=False),
    )
    def _sc_embed(table_hbm, idx_hbm, g_hbm,
                  idx_v, idxT_v, rows_v, g0, g1, w0, w1):
        wid = lax.axis_index("s") * NC + lax.axis_index("c")
        b0 = wid * RPW

        gsem = (g0, g1)
        wsem = (w0, w1)

        pltpu.sync_copy(idx_hbm.at[pl.ds(b0, RPW)], idx_v)
        for h in range(RPW // 16):
            rows16 = 16 * h + lax.iota(jnp.int32, 16)
            for s in range(sch):
                v = plsc.load_gather(
                    idx_v, [rows16, jnp.full((16,), s0 + s, jnp.int32)])
                idxT_v[s, 16 * h:16 * h + 16] = v

        def gather(s, slot):
            pltpu.async_copy(table_hbm.at[idxT_v.at[s]], rows_v.at[slot],
                             gsem[slot])

        def wait_gather(slot):
            pltpu.make_async_copy(table_hbm.at[idxT_v.at[0]], rows_v.at[slot],
                                  gsem[slot]).wait()

        def write(s, slot):
            pltpu.async_copy(rows_v.at[slot], g_hbm.at[s, pl.ds(b0, RPW)],
                             wsem[slot])

        def wait_write(slot):
            pltpu.make_async_copy(rows_v.at[slot], g_hbm.at[0, pl.ds(b0, RPW)],
                                  wsem[slot]).wait()

        gather(0, 0)
        gather(1, 1)
        wait_gather(0)
        write(0, 0)

        def step(p, carry):
            s = 2 * p
            wait_write(0)
            gather(s, 0)
            wait_gather(1)
            write(s - 1, 1)
            wait_write(1)
            gather(s + 1, 1)
            wait_gather(0)
            write(s, 0)
            return carry

        lax.fori_loop(1, sch // 2, step, 0)

        wait_gather(1)
        write(sch - 1, 1)
        wait_write(0)
        wait_write(1)

    return _sc_embed


def _proj_body(tb, w_ref, b_ref, g_ref, *rest):
    y_ref = rest[-1]
    g = g_ref[...].reshape(tb * BATCH, EMBED_DIM).astype(jnp.bfloat16)
    acc = lax.dot_general(
        w_ref[...], g,
        dimension_numbers=(((1,), (1,)), ((), ())),
        preferred_element_type=jnp.float32,
    ) + b_ref[...]
    for t in range(tb):
        y_ref[t] = acc[:, t * BATCH:(t + 1) * BATCH]


def _make_project(s0, sch, first):
    tb = 4 if sch % 4 == 0 else 2
    in_specs = [
        pl.BlockSpec((VOCAB, EMBED_DIM), lambda s: (0, 0)),
        pl.BlockSpec((VOCAB, 1), lambda s: (0, 0)),
        pl.BlockSpec((tb, BATCH, EMBED_DIM), lambda s: (s, 0, 0)),
    ]
    kwargs = {}
    if not first:
        in_specs.append(pl.BlockSpec(memory_space=pl.ANY))
        kwargs["input_output_aliases"] = {3: 0}
    body = functools.partial(_proj_body, tb)
    return pl.pallas_call(
        body,
        grid=(sch // tb,),
        in_specs=in_specs,
        out_specs=pl.BlockSpec((tb, VOCAB, BATCH),
                               lambda s: (s0 // tb + s, 0, 0)),
        out_shape=jax.ShapeDtypeStruct((SEQ, VOCAB, BATCH), jnp.float32),
        compiler_params=pltpu.CompilerParams(
            vmem_limit_bytes=56 * 1024 * 1024),
        **kwargs,
    )


_SC_KERNELS = [_make_sc_embed(s0, sch) for s0, sch in CHUNKS]
_TC_KERNELS = [_make_project(s0, sch, i == 0)
               for i, (s0, sch) in enumerate(CHUNKS)]


@jax.jit
def kernel(x, table, W, b):
    w16 = W.astype(jnp.bfloat16)
    b2d = b.reshape(VOCAB, 1)
    gs = [sck(table, x) for sck in _SC_KERNELS]
    y = _TC_KERNELS[0](w16, b2d, gs[0])
    for i in range(1, len(CHUNKS)):
        y = _TC_KERNELS[i](w16, b2d, gs[i], y)
    return jnp.transpose(y, (2, 0, 1))


# speedup vs baseline: 1.0291x; 1.0291x over previous
import functools

import jax
import jax.numpy as jnp
from jax import lax
from jax.experimental import pallas as pl
from jax.experimental.pallas import tpu as pltpu
from jax.experimental.pallas import tpu_sc as plsc

VOCAB = 1000
EMBED_DIM = 128
BATCH = 1024
SEQ = 50

NC, NS = 2, 16
NW = NC * NS
RPW = BATCH // NW

CHUNKS = ((0, 24), (24, 26))

_SC_MESH = plsc.VectorSubcoreMesh(
    core_axis_name="c", subcore_axis_name="s", num_cores=NC, num_subcores=NS)


def _make_sc_embed(s0, sch):
    @functools.partial(
        pl.kernel,
        out_type=jax.ShapeDtypeStruct((sch, BATCH, EMBED_DIM), jnp.float32),
        mesh=_SC_MESH,
        scratch_types=[
            pltpu.VMEM((sch, 128), jnp.int32),
            pltpu.VMEM((2, RPW, EMBED_DIM), jnp.float32),
            pltpu.SemaphoreType.DMA,
            pltpu.SemaphoreType.DMA,
            pltpu.SemaphoreType.DMA,
            pltpu.SemaphoreType.DMA,
        ],
        compiler_params=pltpu.CompilerParams(
            use_tc_tiling_on_sc=True, needs_layout_passes=False),
    )
    def _sc_embed(table_hbm, idxT_hbm, g_hbm,
                  idxT_v, rows_v, g0, g1, w0, w1):
        wid = lax.axis_index("s") * NC + lax.axis_index("c")
        b0 = wid * RPW

        gsem = (g0, g1)
        wsem = (w0, w1)

        lane0 = (b0 // 128) * 128
        off = b0 - lane0
        pltpu.sync_copy(idxT_hbm.at[pl.ds(s0, sch), pl.ds(lane0, 128)], idxT_v)

        def gather(s, slot):
            pltpu.async_copy(table_hbm.at[idxT_v.at[s, pl.ds(off, RPW)]],
                             rows_v.at[slot], gsem[slot])

        def wait_gather(slot):
            pltpu.make_async_copy(
                table_hbm.at[idxT_v.at[0, pl.ds(off, RPW)]],
                rows_v.at[slot], gsem[slot]).wait()

        def write(s, slot):
            pltpu.async_copy(rows_v.at[slot], g_hbm.at[s, pl.ds(b0, RPW)],
                             wsem[slot])

        def wait_write(slot):
            pltpu.make_async_copy(rows_v.at[slot], g_hbm.at[0, pl.ds(b0, RPW)],
                                  wsem[slot]).wait()

        gather(0, 0)
        gather(1, 1)
        wait_gather(0)
        write(0, 0)

        def step(p, carry):
            s = 2 * p
            wait_write(0)
            gather(s, 0)
            wait_gather(1)
            write(s - 1, 1)
            wait_write(1)
            gather(s + 1, 1)
            wait_gather(0)
            write(s, 0)
            return carry

        lax.fori_loop(1, sch // 2, step, 0)

        wait_gather(1)
        write(sch - 1, 1)
        wait_write(0)
        wait_write(1)

    return _sc_embed


def _proj_body(tb, w_ref, b_ref, g_ref, *rest):
    y_ref = rest[-1]
    g = g_ref[...].reshape(tb * BATCH, EMBED_DIM).astype(jnp.bfloat16)
    acc = lax.dot_general(
        w_ref[...], g,
        dimension_numbers=(((1,), (1,)), ((), ())),
        preferred_element_type=jnp.float32,
    ) + b_ref[...]
    for t in range(tb):
        y_ref[t] = acc[:, t * BATCH:(t + 1) * BATCH]


def _make_project(s0, sch, first):
    tb = 4 if (sch % 4 == 0 and s0 % 4 == 0) else 2
    in_specs = [
        pl.BlockSpec((VOCAB, EMBED_DIM), lambda s: (0, 0)),
        pl.BlockSpec((VOCAB, 1), lambda s: (0, 0)),
        pl.BlockSpec((tb, BATCH, EMBED_DIM), lambda s: (s, 0, 0)),
    ]
    kwargs = {}
    if not first:
        in_specs.append(pl.BlockSpec(memory_space=pl.ANY))
        kwargs["input_output_aliases"] = {3: 0}
    body = functools.partial(_proj_body, tb)
    return pl.pallas_call(
        body,
        grid=(sch // tb,),
        in_specs=in_specs,
        out_specs=pl.BlockSpec((tb, VOCAB, BATCH),
                               lambda s: (s0 // tb + s, 0, 0)),
        out_shape=jax.ShapeDtypeStruct((SEQ, VOCAB, BATCH), jnp.float32),
        compiler_params=pltpu.CompilerParams(
            vmem_limit_bytes=56 * 1024 * 1024),
        **kwargs,
    )


_SC_KERNELS = [_make_sc_embed(s0, sch) for s0, sch in CHUNKS]
_TC_KERNELS = [_make_project(s0, sch, i == 0)
               for i, (s0, sch) in enumerate(CHUNKS)]


@jax.jit
def kernel(x, table, W, b):
    w16 = W.astype(jnp.bfloat16)
    b2d = b.reshape(VOCAB, 1)
    xT = jnp.transpose(x)
    gs = [sck(table, xT) for sck in _SC_KERNELS]
    y = _TC_KERNELS[0](w16, b2d, gs[0])
    for i in range(1, len(CHUNKS)):
        y = _TC_KERNELS[i](w16, b2d, gs[i], y)
    return jnp.transpose(y, (2, 0, 1))
